# Initial kernel scaffold; baseline (speedup 1.0000x reference)
import jax
import jax.numpy as jnp
from jax.experimental import pallas as pl


def kernel(token_ids, table):
    raise NotImplementedError("write your pallas kernel here")


# speedup vs baseline: 1.8860x; 1.8860x over previous
import functools

import jax
import jax.numpy as jnp
from jax import lax
from jax.experimental import pallas as pl
from jax.experimental.pallas import tpu as pltpu
from jax.experimental.pallas import tpu_sc as plsc

_NUM_CORES = 2
_NUM_SUBCORES = 16
_NUM_WORKERS = _NUM_CORES * _NUM_SUBCORES
_LANES = 16

_STREAM_ROWS = 128


def _make_sc_group_sum(total_rows, n_sub, dim):
  groups = total_rows // n_sub
  assert groups % _NUM_WORKERS == 0
  g_per_w = groups // _NUM_WORKERS

  g_chunk = 256
  while g_per_w % g_chunk != 0:
    g_chunk //= 2
  r_chunk = g_chunk * n_sub
  n_steps = g_per_w // g_chunk
  n_streams = pl.cdiv(r_chunk, _STREAM_ROWS)

  mesh = plsc.VectorSubcoreMesh(
      core_axis_name="c", subcore_axis_name="s")

  @functools.partial(
      pl.kernel,
      out_type=jax.ShapeDtypeStruct((groups, dim), jnp.float32),
      mesh=mesh,
      scratch_types=[
          pltpu.VMEM((r_chunk,), jnp.int32),
          pltpu.VMEM((r_chunk, dim), jnp.float32),
          pltpu.VMEM((g_chunk, dim), jnp.float32),
          pltpu.SemaphoreType.DMA,
          pltpu.SemaphoreType.DMA,
      ],
      compiler_params=pltpu.CompilerParams(use_tc_tiling_on_sc=False),
  )
  def k(ids_hbm, table_hbm, out_hbm, idx_v, rows_v, sums_v, gsem, osem):
    wid = lax.axis_index("s") * _NUM_CORES + lax.axis_index("c")
    w_row0 = wid * g_per_w * n_sub

    def step(c, carry):
      del carry
      base = w_row0 + c * r_chunk
      pltpu.sync_copy(ids_hbm.at[pl.ds(base, r_chunk)], idx_v)
      copies = []
      for s in range(n_streams):
        copies.append(pltpu.async_copy(
            table_hbm.at[idx_v.at[pl.ds(s * _STREAM_ROWS, _STREAM_ROWS)]],
            rows_v.at[pl.ds(s * _STREAM_ROWS, _STREAM_ROWS), :],
            gsem))
      for cp in copies:
        cp.wait()

      def pool(g, carry2):
        del carry2
        r = g * n_sub
        for h in range(dim // _LANES):
          d = pl.ds(h * _LANES, _LANES)
          acc = rows_v[r, d]
          for j in range(1, n_sub):
            acc = acc + rows_v[r + j, d]
          sums_v[g, d] = acc
        return 0

      lax.fori_loop(0, g_chunk, pool, 0)
      pltpu.async_copy(
          sums_v,
          out_hbm.at[pl.ds(wid * g_per_w + c * g_chunk, g_chunk), :],
          osem).wait()
      return 0

    lax.fori_loop(0, n_steps, step, 0)

  return k


def _fixup_body(s_ref, ids_ref, row0_ref, o_ref, *, n_sub):
  ids = ids_ref[...]
  npad = jnp.sum((ids == 0).astype(jnp.float32), axis=1, keepdims=True)
  n = float(n_sub) - npad
  empty = n == 0.0
  denom = jnp.where(empty, 1.0, n)
  corrected = s_ref[...] - npad * row0_ref[...]
  o_ref[...] = jnp.where(empty, 0.0, corrected / denom)


def kernel(token_ids, table):
  b, l, n_sub = token_ids.shape
  vocab, dim = table.shape
  groups = b * l

  ids_flat = token_ids.reshape(groups * n_sub)
  sc_sum = _make_sc_group_sum(groups * n_sub, n_sub, dim)
  s = sc_sum(ids_flat, table)

  row0 = lax.slice(table, (0, 0), (1, dim))
  ids2 = token_ids.reshape(groups, n_sub)

  g_blk = 2048
  grid = (pl.cdiv(groups, g_blk),)
  out = pl.pallas_call(
      functools.partial(_fixup_body, n_sub=n_sub),
      grid=grid,
      in_specs=[
          pl.BlockSpec((g_blk, dim), lambda i: (i, 0)),
          pl.BlockSpec((g_blk, n_sub), lambda i: (i, 0)),
          pl.BlockSpec((1, dim), lambda i: (0, 0)),
      ],
      out_specs=pl.BlockSpec((g_blk, dim), lambda i: (i, 0)),
      out_shape=jax.ShapeDtypeStruct((groups, dim), jnp.float32),
  )(s, ids2, row0)

  return out.reshape(b, l, dim)


# speedup vs baseline: 2.1843x; 1.1582x over previous
import functools

import jax
import jax.numpy as jnp
from jax import lax
from jax.experimental import pallas as pl
from jax.experimental.pallas import tpu as pltpu
from jax.experimental.pallas import tpu_sc as plsc

_NUM_CORES = 2
_NUM_SUBCORES = 16
_NUM_WORKERS = _NUM_CORES * _NUM_SUBCORES
_LANES = 16

_STREAM_ROWS = 128


def _make_sc_group_sum(l_dim, b_dim, n_sub, dim):
  groups = l_dim * b_dim
  assert groups % _NUM_WORKERS == 0
  g_per_w = groups // _NUM_WORKERS

  g_chunk = 256
  while g_per_w % g_chunk != 0 or b_dim % g_chunk != 0:
    g_chunk //= 2
  r_chunk = g_chunk * n_sub
  n_steps = g_per_w // g_chunk
  n_streams = pl.cdiv(g_chunk, _STREAM_ROWS)

  mesh = plsc.VectorSubcoreMesh(core_axis_name="c", subcore_axis_name="s")

  @functools.partial(
      pl.kernel,
      out_type=jax.ShapeDtypeStruct((l_dim, b_dim, dim), jnp.float32),
      mesh=mesh,
      scratch_types=[
          pltpu.VMEM((n_sub, g_chunk), jnp.int32),
          pltpu.VMEM((r_chunk, dim), jnp.float32),
          pltpu.VMEM((g_chunk, dim), jnp.float32),
          pltpu.SemaphoreType.DMA,
          pltpu.SemaphoreType.DMA,
      ],
      compiler_params=pltpu.CompilerParams(use_tc_tiling_on_sc=False),
  )
  def k(tt_hbm, table_hbm, out_hbm, idx_v, rows_v, sums_v, gsem, osem):
    wid = lax.axis_index("s") * _NUM_CORES + lax.axis_index("c")
    g_base = wid * g_per_w

    def step(c, carry):
      del carry
      g0 = g_base + c * g_chunk
      l = g0 // b_dim
      b0 = g0 % b_dim
      pltpu.sync_copy(tt_hbm.at[l, :, pl.ds(b0, g_chunk)], idx_v)
      copies = []
      for n in range(n_sub):
        for s in range(n_streams):
          copies.append(pltpu.async_copy(
              table_hbm.at[idx_v.at[n, pl.ds(s * _STREAM_ROWS, _STREAM_ROWS)]],
              rows_v.at[pl.ds(n * g_chunk + s * _STREAM_ROWS, _STREAM_ROWS), :],
              gsem))
      for cp in copies:
        cp.wait()

      def pool(g, carry2):
        del carry2
        for h in range(dim // _LANES):
          d = pl.ds(h * _LANES, _LANES)
          acc = rows_v[g, d]
          for n in range(1, n_sub):
            acc = acc + rows_v[n * g_chunk + g, d]
          sums_v[g, d] = acc
        return 0

      lax.fori_loop(0, g_chunk, pool, 0)
      pltpu.async_copy(
          sums_v, out_hbm.at[l, pl.ds(b0, g_chunk), :], osem).wait()
      return 0

    lax.fori_loop(0, n_steps, step, 0)

  return k


def _fixup_body(s_ref, tt_ref, row0_ref, o_ref, *, n_sub, dim):
  ids = tt_ref[0]
  npad = jnp.sum((ids == 0).astype(jnp.float32), axis=0, keepdims=True)
  n = float(n_sub) - npad
  empty = n == 0.0
  denom = jnp.where(empty, 1.0, n)
  st = jnp.swapaxes(s_ref[0], 0, 1)
  row0t = row0_ref[...].reshape(dim, 1)
  o_ref[0] = jnp.where(empty, 0.0, (st - npad * row0t) / denom)


def kernel(token_ids, table):
  b, l, n_sub = token_ids.shape
  vocab, dim = table.shape

  tt = jnp.transpose(token_ids, (1, 2, 0))
  sc_sum = _make_sc_group_sum(l, b, n_sub, dim)
  s = sc_sum(tt, table)

  row0 = lax.slice(table, (0, 0), (1, dim))

  out_t = pl.pallas_call(
      functools.partial(_fixup_body, n_sub=n_sub, dim=dim),
      grid=(l,),
      in_specs=[
          pl.BlockSpec((1, b, dim), lambda i: (i, 0, 0)),
          pl.BlockSpec((1, n_sub, b), lambda i: (i, 0, 0)),
          pl.BlockSpec((1, dim), lambda i: (0, 0)),
      ],
      out_specs=pl.BlockSpec((1, dim, b), lambda i: (i, 0, 0)),
      out_shape=jax.ShapeDtypeStruct((l, dim, b), jnp.float32),
  )(s, tt, row0)

  return jnp.transpose(out_t, (2, 0, 1))
